# baseline (speedup 1.0000x reference)
import functools

import jax
import jax.numpy as jnp
from jax import lax
from jax.experimental import pallas as pl
from jax.experimental.pallas import tpu as pltpu
from jax.experimental.pallas import tpu_sc as plsc

_B = 16
_N = 2048
_L = 16
_NCHUNK = _N // _L
_NW = 32
_ROWS_PER_W = (_B * _N) // _NW


def _sqrt16(v):
    i = lax.bitcast_convert_type(v, jnp.int32)
    i = jnp.int32(0x5F3759DF) - lax.shift_right_logical(i, 1)
    y = lax.bitcast_convert_type(i, jnp.float32)
    h = jnp.float32(0.5) * v
    for _ in range(3):
        y = y * (jnp.float32(1.5) - h * y * y)
    return v * y


def _sc_body(x_hbm, out_hbm, xs, ys, zs,
             dbuf0, dbuf1, cbuf0, cbuf1, outv):
    dbufs = (dbuf0, dbuf1)
    cbufs = (cbuf0, cbuf1)
    nc = 2
    wid = lax.axis_index("s") * nc + lax.axis_index("c")
    b = wid // 2
    row0 = (wid % 2) * _ROWS_PER_W

    pltpu.sync_copy(x_hbm.at[pl.ds(b * (3 * _N), _N)], xs)
    pltpu.sync_copy(x_hbm.at[pl.ds(b * (3 * _N) + _N, _N)], ys)
    pltpu.sync_copy(x_hbm.at[pl.ds(b * (3 * _N) + 2 * _N, _N)], zs)

    inf16 = jnp.full((_L,), jnp.inf, dtype=jnp.float32)

    def row_group(ii, carry):
        s1, s2 = carry
        base = pl.multiple_of(row0 + ii * _L, _L)
        qxv = xs[pl.ds(base, _L)]
        qyv = ys[pl.ds(base, _L)]
        qzv = zs[pl.ds(base, _L)]
        nr = 2
        for j in range(0, _L, nr):
            qx = [qxv[j + r] for r in range(nr)]
            qy = [qyv[j + r] for r in range(nr)]
            qz = [qzv[j + r] for r in range(nr)]

            def dist_step(g, carry):
                ms = list(carry)
                o = pl.multiple_of(g * _L, _L)
                px = xs[pl.ds(o, _L)]
                py = ys[pl.ds(o, _L)]
                pz = zs[pl.ds(o, _L)]
                for r in range(nr):
                    dx = px - qx[r]
                    dy = py - qy[r]
                    dz = pz - qz[r]
                    d = dx * dx + dy * dy + dz * dz
                    dbufs[r][pl.ds(o, _L)] = d
                    ms[r] = jnp.minimum(ms[r], d)
                return tuple(ms)

            ms = lax.fori_loop(0, _NCHUNK, dist_step, (inf16,) * nr)
            ts = [jnp.max(ms[r]) for r in range(nr)]

            def compress_step(g, carry):
                os_ = list(carry)
                o = pl.multiple_of(g * _L, _L)
                for r in range(nr):
                    d = dbufs[r][pl.ds(o, _L)]
                    msk = d <= ts[r]
                    plsc.store_compressed(
                        cbufs[r].at[pl.ds(os_[r], _L)], d, mask=msk)
                    os_[r] = os_[r] + plsc.all_reduce_population_count(msk)[0]
                return tuple(os_)

            zero = jnp.int32(0)
            os_ = lax.fori_loop(0, _NCHUNK, compress_step, (zero,) * nr)
            for r in range(nr):
                cbufs[r][pl.ds(os_[r], _L)] = inf16

            for r in range(nr):
                cb = cbufs[r]

                def merge_step(g, best, cb=cb):
                    c = cb[pl.ds(g * _L, _L)]
                    return jnp.sort(jnp.minimum(best, jnp.flip(jnp.sort(c))))

                nv = (os_[r] + (_L - 1)) // _L
                best = lax.fori_loop(0, nv, merge_step, inf16)
                dist = _sqrt16(best + jnp.float32(1e-12))
                tot = jnp.sum(dist)
                mn = jnp.min(dist)
                m = (tot - mn) * jnp.float32(1.0 / 15.0)
                s1 = s1 + m
                s2 = s2 + m * m
        return s1, s2

    s1, s2 = lax.fori_loop(
        0, _ROWS_PER_W // _L, row_group,
        (jnp.float32(0.0), jnp.float32(0.0)))

    lanes = lax.iota(jnp.int32, _L)
    res = jnp.where(lanes == 0, s1, jnp.where(lanes == 1, s2, 0.0))
    res = res.astype(jnp.float32)
    outv[...] = res
    pltpu.sync_copy(outv, out_hbm.at[wid])


@jax.jit
def kernel(x):
    xt = jnp.transpose(x, (0, 2, 1)).reshape(-1)
    mesh = plsc.VectorSubcoreMesh(core_axis_name="c", subcore_axis_name="s")
    run = functools.partial(
        pl.kernel,
        out_type=jax.ShapeDtypeStruct((_NW, _L), jnp.float32),
        mesh=mesh,
        scratch_types=[
            pltpu.VMEM((_N,), jnp.float32),
            pltpu.VMEM((_N,), jnp.float32),
            pltpu.VMEM((_N,), jnp.float32),
            pltpu.VMEM((_N,), jnp.float32),
            pltpu.VMEM((_N,), jnp.float32),
            pltpu.VMEM((_N + _L,), jnp.float32),
            pltpu.VMEM((_N + _L,), jnp.float32),
            pltpu.VMEM((_L,), jnp.float32),
        ],
        compiler_params=pltpu.CompilerParams(needs_layout_passes=False),
    )(_sc_body)
    parts = run(xt)
    parts = parts.reshape(_B, 2, _L)
    s1 = parts[:, 0, 0] + parts[:, 1, 0]
    s2 = parts[:, 0, 1] + parts[:, 1, 1]
    n = jnp.float32(_N)
    var = (s2 - s1 * s1 / n) / (n - 1.0)
    return jnp.sqrt(jnp.maximum(var, 0.0))


# speedup vs baseline: 1.5038x; 1.5038x over previous
import functools

import jax
import jax.numpy as jnp
from jax import lax
from jax.experimental import pallas as pl
from jax.experimental.pallas import tpu as pltpu
from jax.experimental.pallas import tpu_sc as plsc

_B = 16
_N = 2048
_L = 16
_NCHUNK = _N // _L
_NW = 32
_ROWS_PER_W = (_B * _N) // _NW


def _sqrt16(v):
    i = lax.bitcast_convert_type(v, jnp.int32)
    i = jnp.int32(0x5F3759DF) - lax.shift_right_logical(i, 1)
    y = lax.bitcast_convert_type(i, jnp.float32)
    h = jnp.float32(0.5) * v
    for _ in range(3):
        y = y * (jnp.float32(1.5) - h * y * y)
    return v * y


def _sc_body(x_hbm, out_hbm, xs, ys, zs, dbufa, dbufb, cbufa, cbufb, outv):
    nc = 2
    wid = lax.axis_index("s") * nc + lax.axis_index("c")
    b = wid // 2
    row0 = (wid % 2) * _ROWS_PER_W

    pltpu.sync_copy(x_hbm.at[pl.ds(b * (3 * _N), _N)], xs)
    pltpu.sync_copy(x_hbm.at[pl.ds(b * (3 * _N) + _N, _N)], ys)
    pltpu.sync_copy(x_hbm.at[pl.ds(b * (3 * _N) + 2 * _N, _N)], zs)

    inf16 = jnp.full((_L,), jnp.inf, dtype=jnp.float32)

    def row_group(ii, carry):
        s1, s2 = carry
        base = pl.multiple_of(row0 + ii * _L, _L)
        qxv = xs[pl.ds(base, _L)]
        qyv = ys[pl.ds(base, _L)]
        qzv = zs[pl.ds(base, _L)]
        for j in range(0, _L, 2):
            qxa, qxb = qxv[j], qxv[j + 1]
            qya, qyb = qyv[j], qyv[j + 1]
            qza, qzb = qzv[j], qzv[j + 1]

            def dist_step(g, carry):
                ma, mb = carry
                o = pl.multiple_of(g * _L, _L)
                px = xs[pl.ds(o, _L)]
                py = ys[pl.ds(o, _L)]
                pz = zs[pl.ds(o, _L)]
                dxa = px - qxa
                dya = py - qya
                dza = pz - qza
                da = dxa * dxa + dya * dya + dza * dza
                dxb = px - qxb
                dyb = py - qyb
                dzb = pz - qzb
                db = dxb * dxb + dyb * dyb + dzb * dzb
                dbufa[pl.ds(o, _L)] = da
                dbufb[pl.ds(o, _L)] = db
                return jnp.minimum(ma, da), jnp.minimum(mb, db)

            ma, mb = lax.fori_loop(0, _NCHUNK, dist_step, (inf16, inf16))
            ta = jnp.max(ma)
            tb = jnp.max(mb)

            def compress_step(g, carry):
                oa, ob = carry
                o = pl.multiple_of(g * _L, _L)
                da = dbufa[pl.ds(o, _L)]
                db = dbufb[pl.ds(o, _L)]
                mska = da <= ta
                mskb = db <= tb
                plsc.store_compressed(cbufa.at[pl.ds(oa, _L)], da, mask=mska)
                plsc.store_compressed(cbufb.at[pl.ds(ob, _L)], db, mask=mskb)
                ca = plsc.all_reduce_population_count(mska)[0]
                cb = plsc.all_reduce_population_count(mskb)[0]
                return oa + ca, ob + cb

            zero = jnp.int32(0)
            oa, ob = lax.fori_loop(0, _NCHUNK, compress_step, (zero, zero))
            cbufa[pl.ds(oa, _L)] = inf16
            cbufb[pl.ds(ob, _L)] = inf16

            def merge_step_a(g, best):
                c = cbufa[pl.ds(g * _L, _L)]
                return jnp.sort(jnp.minimum(best, jnp.flip(jnp.sort(c))))

            def merge_step_b(g, best):
                c = cbufb[pl.ds(g * _L, _L)]
                return jnp.sort(jnp.minimum(best, jnp.flip(jnp.sort(c))))

            nva = (oa + (_L - 1)) // _L
            nvb = (ob + (_L - 1)) // _L
            besta = lax.fori_loop(0, nva, merge_step_a, inf16)
            bestb = lax.fori_loop(0, nvb, merge_step_b, inf16)

            for best in (besta, bestb):
                dist = _sqrt16(best + jnp.float32(1e-12))
                tot = jnp.sum(dist)
                mn = jnp.min(dist)
                m = (tot - mn) * jnp.float32(1.0 / 15.0)
                s1 = s1 + m
                s2 = s2 + m * m
        return s1, s2

    s1, s2 = lax.fori_loop(
        0, _ROWS_PER_W // _L, row_group,
        (jnp.float32(0.0), jnp.float32(0.0)))

    lanes = lax.iota(jnp.int32, _L)
    res = jnp.where(lanes == 0, s1, jnp.where(lanes == 1, s2, 0.0))
    res = res.astype(jnp.float32)
    outv[...] = res
    pltpu.sync_copy(outv, out_hbm.at[wid])


@jax.jit
def kernel(x):
    xt = jnp.transpose(x, (0, 2, 1)).reshape(-1)
    mesh = plsc.VectorSubcoreMesh(core_axis_name="c", subcore_axis_name="s")
    run = functools.partial(
        pl.kernel,
        out_type=jax.ShapeDtypeStruct((_NW, _L), jnp.float32),
        mesh=mesh,
        scratch_types=[
            pltpu.VMEM((_N,), jnp.float32),
            pltpu.VMEM((_N,), jnp.float32),
            pltpu.VMEM((_N,), jnp.float32),
            pltpu.VMEM((_N,), jnp.float32),
            pltpu.VMEM((_N,), jnp.float32),
            pltpu.VMEM((_N + _L,), jnp.float32),
            pltpu.VMEM((_N + _L,), jnp.float32),
            pltpu.VMEM((_L,), jnp.float32),
        ],
        compiler_params=pltpu.CompilerParams(needs_layout_passes=False),
    )(_sc_body)
    parts = run(xt)
    parts = parts.reshape(_B, 2, _L)
    s1 = parts[:, 0, 0] + parts[:, 1, 0]
    s2 = parts[:, 0, 1] + parts[:, 1, 1]
    n = jnp.float32(_N)
    var = (s2 - s1 * s1 / n) / (n - 1.0)
    return jnp.sqrt(jnp.maximum(var, 0.0))


# speedup vs baseline: 1.5643x; 1.0402x over previous
import functools

import jax
import jax.numpy as jnp
from jax import lax
from jax.experimental import pallas as pl
from jax.experimental.pallas import tpu as pltpu
from jax.experimental.pallas import tpu_sc as plsc

_B = 16
_N = 2048
_L = 16
_NCHUNK = _N // _L
_NW = 32
_ROWS_PER_W = (_B * _N) // _NW


def _sqrt16(v):
    i = lax.bitcast_convert_type(v, jnp.int32)
    i = jnp.int32(0x5F3759DF) - lax.shift_right_logical(i, 1)
    y = lax.bitcast_convert_type(i, jnp.float32)
    h = jnp.float32(0.5) * v
    for _ in range(3):
        y = y * (jnp.float32(1.5) - h * y * y)
    return v * y


def _sc_body(x_hbm, out_hbm, xs, ys, zs, dbufa, dbufb, cbufa, cbufb, outv):
    nc = 2
    wid = lax.axis_index("s") * nc + lax.axis_index("c")
    b = wid // 2
    row0 = (wid % 2) * _ROWS_PER_W

    pltpu.sync_copy(x_hbm.at[pl.ds(b * (3 * _N), _N)], xs)
    pltpu.sync_copy(x_hbm.at[pl.ds(b * (3 * _N) + _N, _N)], ys)
    pltpu.sync_copy(x_hbm.at[pl.ds(b * (3 * _N) + 2 * _N, _N)], zs)

    inf16 = jnp.full((_L,), jnp.inf, dtype=jnp.float32)

    def row_group(ii, carry):
        s1, s2 = carry
        base = pl.multiple_of(row0 + ii * _L, _L)
        qxv = xs[pl.ds(base, _L)]
        qyv = ys[pl.ds(base, _L)]
        qzv = zs[pl.ds(base, _L)]
        for j in range(0, _L, 2):
            qxa, qxb = qxv[j], qxv[j + 1]
            qya, qyb = qyv[j], qyv[j + 1]
            qza, qzb = qzv[j], qzv[j + 1]

            def dist_step(g, carry):
                ma, mb = carry
                o = pl.multiple_of(g * _L, _L)
                px = xs[pl.ds(o, _L)]
                py = ys[pl.ds(o, _L)]
                pz = zs[pl.ds(o, _L)]
                dxa = px - qxa
                dya = py - qya
                dza = pz - qza
                da = dxa * dxa + dya * dya + dza * dza
                dxb = px - qxb
                dyb = py - qyb
                dzb = pz - qzb
                db = dxb * dxb + dyb * dyb + dzb * dzb
                dbufa[pl.ds(o, _L)] = da
                dbufb[pl.ds(o, _L)] = db
                return jnp.minimum(ma, da), jnp.minimum(mb, db)

            ma, mb = lax.fori_loop(0, _NCHUNK, dist_step, (inf16, inf16))
            ta = jnp.max(ma)
            tb = jnp.max(mb)

            def compress_step(g, carry):
                oa, ob = carry
                o = pl.multiple_of(g * _L, _L)
                da = dbufa[pl.ds(o, _L)]
                db = dbufb[pl.ds(o, _L)]
                mska = da <= ta
                mskb = db <= tb
                ca = plsc.all_reduce_population_count(mska)[0]
                cb = plsc.all_reduce_population_count(mskb)[0]
                plsc.store_compressed(cbufa.at[pl.ds(oa, _L)], da, mask=mska)
                plsc.store_compressed(cbufb.at[pl.ds(ob, _L)], db, mask=mskb)
                return oa + ca, ob + cb

            zero = jnp.int32(0)
            oa, ob = lax.fori_loop(0, _NCHUNK, compress_step, (zero, zero))
            cbufa[pl.ds(oa, _L)] = inf16
            cbufb[pl.ds(ob, _L)] = inf16

            nva = (oa + (_L - 1)) // _L
            nvb = (ob + (_L - 1)) // _L

            def merge_step(g, carry):
                ba, bb = carry
                c_a = cbufa[pl.ds(g * _L, _L)]
                c_b = cbufb[pl.ds(g * _L, _L)]
                na = jnp.sort(jnp.minimum(ba, jnp.flip(jnp.sort(c_a))))
                nb = jnp.sort(jnp.minimum(bb, jnp.flip(jnp.sort(c_b))))
                ba = jnp.where(g < nva, na, ba)
                bb = jnp.where(g < nvb, nb, bb)
                return ba, bb

            besta, bestb = lax.fori_loop(
                0, jnp.maximum(nva, nvb), merge_step, (inf16, inf16))

            for best in (besta, bestb):
                dist = _sqrt16(best + jnp.float32(1e-12))
                tot = jnp.sum(dist)
                mn = jnp.min(dist)
                m = (tot - mn) * jnp.float32(1.0 / 15.0)
                s1 = s1 + m
                s2 = s2 + m * m
        return s1, s2

    s1, s2 = lax.fori_loop(
        0, _ROWS_PER_W // _L, row_group,
        (jnp.float32(0.0), jnp.float32(0.0)))

    lanes = lax.iota(jnp.int32, _L)
    res = jnp.where(lanes == 0, s1, jnp.where(lanes == 1, s2, 0.0))
    res = res.astype(jnp.float32)
    outv[...] = res
    pltpu.sync_copy(outv, out_hbm.at[wid])


@jax.jit
def kernel(x):
    xt = jnp.transpose(x, (0, 2, 1)).reshape(-1)
    mesh = plsc.VectorSubcoreMesh(core_axis_name="c", subcore_axis_name="s")
    run = functools.partial(
        pl.kernel,
        out_type=jax.ShapeDtypeStruct((_NW, _L), jnp.float32),
        mesh=mesh,
        scratch_types=[
            pltpu.VMEM((_N,), jnp.float32),
            pltpu.VMEM((_N,), jnp.float32),
            pltpu.VMEM((_N,), jnp.float32),
            pltpu.VMEM((_N,), jnp.float32),
            pltpu.VMEM((_N,), jnp.float32),
            pltpu.VMEM((_N + _L,), jnp.float32),
            pltpu.VMEM((_N + _L,), jnp.float32),
            pltpu.VMEM((_L,), jnp.float32),
        ],
        compiler_params=pltpu.CompilerParams(needs_layout_passes=False),
    )(_sc_body)
    parts = run(xt)
    parts = parts.reshape(_B, 2, _L)
    s1 = parts[:, 0, 0] + parts[:, 1, 0]
    s2 = parts[:, 0, 1] + parts[:, 1, 1]
    n = jnp.float32(_N)
    var = (s2 - s1 * s1 / n) / (n - 1.0)
    return jnp.sqrt(jnp.maximum(var, 0.0))
